# baseline (speedup 1.0000x reference)
import functools

import jax
import jax.numpy as jnp
from jax import lax
from jax.experimental import pallas as pl
from jax.experimental.pallas import tpu as pltpu
from jax.experimental.pallas import tpu_sc as plsc

B, S, V, D = 1024, 200, 100000, 128
ROWS = 10000
NB = V // ROWS
T16 = 16


def _fused_table_body(ph_ref, sl_ref, pl_ref_, st_ref, pt_ref, w_ref, b_ref,
                      out_ref):
    w1 = w_ref[0:D, :].astype(jnp.bfloat16)
    sproj = jnp.dot(st_ref[:], w_ref[D:2 * D, :],
                    preferred_element_type=jnp.float32)
    pproj = jnp.dot(pt_ref[:], w_ref[2 * D:3 * D, :],
                    preferred_element_type=jnp.float32)
    acc = jnp.dot(ph_ref[:], w1, preferred_element_type=jnp.float32)
    sidx = sl_ref[0, 0, :]
    pidx = pl_ref_[0, 0, :]
    iota = lax.broadcasted_iota(jnp.int32, (ROWS, D), 1)
    oh = ((sidx[:, None] == iota) | ((pidx[:, None] + T16) == iota))
    proj = jnp.concatenate(
        [sproj, pproj, jnp.zeros((D - 2 * T16, D), jnp.float32)],
        axis=0).astype(jnp.bfloat16)
    acc = acc + jnp.dot(oh.astype(jnp.bfloat16), proj,
                        preferred_element_type=jnp.float32)
    out_ref[:] = acc + b_ref[:]


def _build_fused_table(phoneme_table, sl3, pl3, st16, pt16, W, b2):
    return pl.pallas_call(
        _fused_table_body,
        grid=(NB,),
        in_specs=[
            pl.BlockSpec((ROWS, D), lambda i: (i, 0)),
            pl.BlockSpec((1, 1, ROWS), lambda i: (i, 0, 0)),
            pl.BlockSpec((1, 1, ROWS), lambda i: (i, 0, 0)),
            pl.BlockSpec((T16, D), lambda i: (0, 0)),
            pl.BlockSpec((T16, D), lambda i: (0, 0)),
            pl.BlockSpec((3 * D, D), lambda i: (0, 0)),
            pl.BlockSpec((1, D), lambda i: (0, 0)),
        ],
        out_specs=pl.BlockSpec((ROWS, D), lambda i: (i, 0)),
        out_shape=jax.ShapeDtypeStruct((V, D), jnp.float32),
    )(phoneme_table, sl3, pl3, st16, pt16, W, b2)


NBUF = 5


def _make_sc_gather(total, nw):
    per_w = total // nw
    k = per_w // 128
    ngroups = k // NBUF
    mesh = plsc.VectorSubcoreMesh(core_axis_name="c", subcore_axis_name="s")

    @functools.partial(
        pl.kernel,
        mesh=mesh,
        out_type=jax.ShapeDtypeStruct((total, D), jnp.float32),
        scratch_types=[
            pltpu.VMEM((k, 128), jnp.int32),
            pltpu.VMEM((NBUF, 128, D), jnp.float32),
        ] + [pltpu.SemaphoreType.DMA] * (2 * NBUF),
    )
    def gather(idx_hbm, table_hbm, out_hbm, idx_v, rows_v, *sems):
        sem_g = sems[:NBUF]
        sem_o = sems[NBUF:]
        nc = lax.axis_size("c")
        wid = lax.axis_index("s") * nc + lax.axis_index("c")
        pltpu.sync_copy(idx_hbm.at[wid], idx_v)
        base = wid * per_w

        def gather_cp(j, s):
            return pltpu.make_async_copy(
                table_hbm.at[idx_v.at[j]], rows_v.at[s], sem_g[s])

        def out_cp(j, s):
            return pltpu.make_async_copy(
                rows_v.at[s], out_hbm.at[pl.ds(base + j * 128, 128)],
                sem_o[s])

        def body(t, carry):
            for s in range(NBUF):
                j = t * NBUF + s

                @pl.when(t > 0)
                def _():
                    out_cp(j - NBUF, s).wait()

                gather_cp(j, s).start()
            for s in range(NBUF):
                j = t * NBUF + s
                gather_cp(j, s).wait()
                out_cp(j, s).start()
            return carry

        lax.fori_loop(0, ngroups, body, 0)
        for s in range(NBUF):
            out_cp((ngroups - 1) * NBUF + s, s).wait()

    return gather


def kernel(phoneme_indices, phoneme_table, sutra_table, position_table,
           sutra_lookup, position_lookup, W, b):
    sl3 = sutra_lookup.astype(jnp.int32).reshape(NB, 1, ROWS)
    pl3 = position_lookup.astype(jnp.int32).reshape(NB, 1, ROWS)
    st16 = jnp.pad(sutra_table, ((0, T16 - sutra_table.shape[0]), (0, 0)))
    pt16 = jnp.pad(position_table, ((0, T16 - position_table.shape[0]), (0, 0)))
    b2 = b.reshape(1, D)

    fused = _build_fused_table(phoneme_table.astype(jnp.bfloat16),
                               sl3, pl3, st16, pt16, W, b2)

    info = plsc.get_sparse_core_info()
    nw = info.num_cores * info.num_subcores
    total = B * S
    idx = phoneme_indices.astype(jnp.int32).reshape(nw, total // nw // 128, 128)
    out = _make_sc_gather(total, nw)(idx, fused)
    return out.reshape(B, S, D)


# speedup vs baseline: 1.1472x; 1.1472x over previous
import functools

import jax
import jax.numpy as jnp
from jax import lax
from jax.experimental import pallas as pl
from jax.experimental.pallas import tpu as pltpu
from jax.experimental.pallas import tpu_sc as plsc

B, S, V, D = 1024, 200, 100000, 128
ROWS = 10000
NB = V // ROWS
T16 = 16


def _fused_table_body(ph_ref, sl_ref, pl_ref_, st_ref, pt_ref, w_ref, b_ref,
                      out_ref):
    w1 = w_ref[0:D, :]
    sproj = jnp.dot(st_ref[:], w_ref[D:2 * D, :],
                    preferred_element_type=jnp.float32)
    pproj = jnp.dot(pt_ref[:], w_ref[2 * D:3 * D, :],
                    preferred_element_type=jnp.float32)
    acc = jnp.dot(ph_ref[:], w1, preferred_element_type=jnp.float32)
    sidx = sl_ref[0, 0, :]
    pidx = pl_ref_[0, 0, :]
    iota = lax.broadcasted_iota(jnp.int32, (ROWS, D), 1)
    oh = ((sidx[:, None] == iota) | ((pidx[:, None] + T16) == iota))
    proj = jnp.concatenate(
        [sproj, pproj, jnp.zeros((D - 2 * T16, D), jnp.float32)], axis=0)
    acc = acc + jnp.dot(oh.astype(jnp.float32), proj,
                        preferred_element_type=jnp.float32)
    out_ref[:] = acc + b_ref[:]


def _build_fused_table(phoneme_table, sl3, pl3, st16, pt16, W, b2):
    return pl.pallas_call(
        _fused_table_body,
        grid=(NB,),
        in_specs=[
            pl.BlockSpec((ROWS, D), lambda i: (i, 0)),
            pl.BlockSpec((1, 1, ROWS), lambda i: (i, 0, 0)),
            pl.BlockSpec((1, 1, ROWS), lambda i: (i, 0, 0)),
            pl.BlockSpec((T16, D), lambda i: (0, 0)),
            pl.BlockSpec((T16, D), lambda i: (0, 0)),
            pl.BlockSpec((3 * D, D), lambda i: (0, 0)),
            pl.BlockSpec((1, D), lambda i: (0, 0)),
        ],
        out_specs=pl.BlockSpec((ROWS, D), lambda i: (i, 0)),
        out_shape=jax.ShapeDtypeStruct((V, D), jnp.float32),
    )(phoneme_table, sl3, pl3, st16, pt16, W, b2)


NBUF = 10
CH = 64


def _make_sc_gather(total, nw):
    per_w = total // nw
    k = per_w // CH
    ngroups = k // NBUF
    mesh = plsc.VectorSubcoreMesh(core_axis_name="c", subcore_axis_name="s")

    @functools.partial(
        pl.kernel,
        mesh=mesh,
        out_type=jax.ShapeDtypeStruct((total, D), jnp.float32),
        scratch_types=[
            pltpu.VMEM((k, CH), jnp.int32),
            pltpu.VMEM((NBUF, CH, D), jnp.float32),
        ] + [pltpu.SemaphoreType.DMA] * (2 * NBUF),
    )
    def gather(idx_hbm, table_hbm, out_hbm, idx_v, rows_v, *sems):
        sem_g = sems[:NBUF]
        sem_o = sems[NBUF:]
        nc = lax.axis_size("c")
        wid = lax.axis_index("s") * nc + lax.axis_index("c")
        pltpu.sync_copy(idx_hbm.at[wid], idx_v)
        base = wid * per_w

        def gather_cp(j, s):
            return pltpu.make_async_copy(
                table_hbm.at[idx_v.at[j]], rows_v.at[s], sem_g[s])

        def out_cp(j, s):
            return pltpu.make_async_copy(
                rows_v.at[s], out_hbm.at[pl.ds(base + j * CH, CH)],
                sem_o[s])

        def body(t, carry):
            for s in range(NBUF):
                j = t * NBUF + s

                @pl.when(t > 0)
                def _():
                    out_cp(j - NBUF, s).wait()

                gather_cp(j, s).start()
            for s in range(NBUF):
                j = t * NBUF + s
                gather_cp(j, s).wait()
                out_cp(j, s).start()
            return carry

        lax.fori_loop(0, ngroups, body, 0)
        for s in range(NBUF):
            out_cp((ngroups - 1) * NBUF + s, s).wait()

    return gather


def kernel(phoneme_indices, phoneme_table, sutra_table, position_table,
           sutra_lookup, position_lookup, W, b):
    sl3 = sutra_lookup.astype(jnp.int32).reshape(NB, 1, ROWS)
    pl3 = position_lookup.astype(jnp.int32).reshape(NB, 1, ROWS)
    st16 = jnp.pad(sutra_table, ((0, T16 - sutra_table.shape[0]), (0, 0)))
    pt16 = jnp.pad(position_table, ((0, T16 - position_table.shape[0]), (0, 0)))
    b2 = b.reshape(1, D)

    fused = _build_fused_table(phoneme_table, sl3, pl3, st16, pt16, W, b2)

    info = plsc.get_sparse_core_info()
    nw = info.num_cores * info.num_subcores
    total = B * S
    idx = phoneme_indices.astype(jnp.int32).reshape(nw, total // nw // CH, CH)
    out = _make_sc_gather(total, nw)(idx, fused)
    return out.reshape(B, S, D)


# speedup vs baseline: 1.1636x; 1.0143x over previous
import functools

import jax
import jax.numpy as jnp
from jax import lax
from jax.experimental import pallas as pl
from jax.experimental.pallas import tpu as pltpu
from jax.experimental.pallas import tpu_sc as plsc

B, S, V, D = 1024, 200, 100000, 128
ROWS = 20000
NB = V // ROWS
T16 = 16


def _fused_table_body(ph_ref, sl_ref, pl_ref_, st_ref, pt_ref, w_ref, b_ref,
                      out_ref):
    w1 = w_ref[0:D, :]
    sproj = jnp.dot(st_ref[:], w_ref[D:2 * D, :],
                    preferred_element_type=jnp.float32)
    pproj = jnp.dot(pt_ref[:], w_ref[2 * D:3 * D, :],
                    preferred_element_type=jnp.float32)
    acc = jnp.dot(ph_ref[:], w1, preferred_element_type=jnp.float32)
    sidx = sl_ref[0, 0, :]
    pidx = pl_ref_[0, 0, :]
    iota = lax.broadcasted_iota(jnp.int32, (ROWS, D), 1)
    oh = ((sidx[:, None] == iota) | ((pidx[:, None] + T16) == iota))
    proj = jnp.concatenate(
        [sproj, jnp.zeros((T16 - 15, D), jnp.float32),
         pproj, jnp.zeros((D - T16 - 11, D), jnp.float32)], axis=0)
    acc = acc + jnp.dot(oh.astype(jnp.float32), proj,
                        preferred_element_type=jnp.float32)
    out_ref[:] = acc + b_ref[:]


def _build_fused_table(phoneme_table, sl3, pl3, st16, pt16, W, b2):
    return pl.pallas_call(
        _fused_table_body,
        grid=(NB,),
        in_specs=[
            pl.BlockSpec((ROWS, D), lambda i: (i, 0)),
            pl.BlockSpec((1, 1, ROWS), lambda i: (i, 0, 0)),
            pl.BlockSpec((1, 1, ROWS), lambda i: (i, 0, 0)),
            pl.BlockSpec((15, D), lambda i: (0, 0)),
            pl.BlockSpec((11, D), lambda i: (0, 0)),
            pl.BlockSpec((3 * D, D), lambda i: (0, 0)),
            pl.BlockSpec((1, D), lambda i: (0, 0)),
        ],
        out_specs=pl.BlockSpec((ROWS, D), lambda i: (i, 0)),
        out_shape=jax.ShapeDtypeStruct((V, D), jnp.float32),
    )(phoneme_table, sl3, pl3, st16, pt16, W, b2)


NBUF = 10
CH = 64


def _make_sc_gather(total, nw):
    per_w = total // nw
    k = per_w // CH
    ngroups = k // NBUF
    mesh = plsc.VectorSubcoreMesh(core_axis_name="c", subcore_axis_name="s")

    @functools.partial(
        pl.kernel,
        mesh=mesh,
        out_type=jax.ShapeDtypeStruct((total, D), jnp.float32),
        scratch_types=[
            pltpu.VMEM((k, CH), jnp.int32),
            pltpu.VMEM((NBUF, CH, D), jnp.float32),
        ] + [pltpu.SemaphoreType.DMA] * (2 * NBUF),
    )
    def gather(idx_hbm, table_hbm, out_hbm, idx_v, rows_v, *sems):
        sem_g = sems[:NBUF]
        sem_o = sems[NBUF:]
        nc = lax.axis_size("c")
        wid = lax.axis_index("s") * nc + lax.axis_index("c")
        pltpu.sync_copy(idx_hbm.at[wid], idx_v)
        base = wid * per_w

        def gather_cp(j, s):
            return pltpu.make_async_copy(
                table_hbm.at[idx_v.at[j]], rows_v.at[s], sem_g[s])

        def out_cp(j, s):
            return pltpu.make_async_copy(
                rows_v.at[s], out_hbm.at[pl.ds(base + j * CH, CH)],
                sem_o[s])

        def body(t, carry):
            for s in range(NBUF):
                j = t * NBUF + s

                @pl.when(t > 0)
                def _():
                    out_cp(j - NBUF, s).wait()

                gather_cp(j, s).start()
            for s in range(NBUF):
                j = t * NBUF + s
                gather_cp(j, s).wait()
                out_cp(j, s).start()
            return carry

        lax.fori_loop(0, ngroups, body, 0)
        for s in range(NBUF):
            out_cp((ngroups - 1) * NBUF + s, s).wait()

    return gather


def kernel(phoneme_indices, phoneme_table, sutra_table, position_table,
           sutra_lookup, position_lookup, W, b):
    sl3 = sutra_lookup.astype(jnp.int32).reshape(NB, 1, ROWS)
    pl3 = position_lookup.astype(jnp.int32).reshape(NB, 1, ROWS)
    b2 = b.reshape(1, D)

    fused = _build_fused_table(phoneme_table, sl3, pl3, sutra_table,
                               position_table, W, b2)

    info = plsc.get_sparse_core_info()
    nw = info.num_cores * info.num_subcores
    total = B * S
    idx = phoneme_indices.astype(jnp.int32).reshape(nw, total // nw // CH, CH)
    out = _make_sc_gather(total, nw)(idx, fused)
    return out.reshape(B, S, D)
